# baseline (speedup 1.0000x reference)
import jax
import jax.numpy as jnp
from jax import lax
from jax.experimental import pallas as pl
from jax.experimental.pallas import tpu as pltpu

SIZE = 100000
INPUT_SIZE = 128
BR = 4000
NBLK = SIZE // BR


def _tc_body(inp_ref, blk_ref, out_ref, win_ref, best_ref):
    i = pl.program_id(0)
    blk = blk_ref[...]
    inp = inp_ref[...]
    ov = jnp.dot(blk, inp, preferred_element_type=jnp.float32)
    out_ref[...] = ov

    rows = i * BR + lax.broadcasted_iota(jnp.int32, (BR, 1), 0)
    key = (ov.astype(jnp.int32) << 17) | (131071 - rows)
    blk_best = jnp.max(key)

    @pl.when(i == 0)
    def _init():
        best_ref[0] = blk_best

    @pl.when(i > 0)
    def _upd():
        best_ref[0] = jnp.maximum(best_ref[0], blk_best)

    @pl.when(i == NBLK - 1)
    def _fin():
        win_ref[0] = 131071 - (best_ref[0] & 131071)


def kernel(input_array, connected):
    inp = input_array.astype(jnp.float32).reshape(INPUT_SIZE, 1)
    overlaps2d, winner1 = pl.pallas_call(
        _tc_body,
        grid=(NBLK,),
        in_specs=[
            pl.BlockSpec((INPUT_SIZE, 1), lambda i: (0, 0)),
            pl.BlockSpec((BR, INPUT_SIZE), lambda i: (i, 0)),
        ],
        out_specs=[
            pl.BlockSpec((BR, 1), lambda i: (i, 0)),
            pl.BlockSpec(memory_space=pltpu.SMEM),
        ],
        out_shape=[
            jax.ShapeDtypeStruct((SIZE, 1), jnp.float32),
            jax.ShapeDtypeStruct((1,), jnp.int32),
        ],
        scratch_shapes=[pltpu.SMEM((1,), jnp.int32)],
    )(inp, connected)
    return overlaps2d.reshape(SIZE), winner1[0]
